# baseline (speedup 1.0000x reference)
import functools

import jax
import jax.numpy as jnp
from jax import lax
from jax.experimental import pallas as pl
from jax.experimental.pallas import tpu as pltpu
from jax.experimental.pallas import tpu_sc as plsc

NUM_E = 1_000_000
B = 16384

_NC, _NS = 2, 16
_NW = _NC * _NS
_BPW = B // _NW
_IROWS = _BPW // 128

_UNROLL = 16
_CH = 31264
_NV = _CH // 16
_SUBV = 977
_NBS = 61
_LAST_BASE = 31 * _CH
_LAST_VALID = NUM_E - _LAST_BASE
_LAST_SUB1 = _LAST_VALID - _SUBV * 16

_LN2 = 0.6931471805599453

_sc_mesh = plsc.VectorSubcoreMesh(core_axis_name="c", subcore_axis_name="s")


def _bfly_max(x):
    for sh in (8, 4, 2, 1):
        perm = lax.iota(jnp.int32, 16) ^ sh
        x = jnp.maximum(x, x.at[perm].get(mode="promise_in_bounds"))
    return x


def _bfly_sum(x):
    for sh in (8, 4, 2, 1):
        perm = lax.iota(jnp.int32, 16) ^ sh
        x = x + x.at[perm].get(mode="promise_in_bounds")
    return x


@functools.partial(
    pl.kernel,
    mesh=_sc_mesh,
    out_type=(
        jax.ShapeDtypeStruct((B,), jnp.float32),
        jax.ShapeDtypeStruct((2, 32), jnp.float32),
    ),
    scratch_types=[
        pltpu.VMEM((_BPW,), jnp.int32),
        pltpu.VMEM((_BPW,), jnp.float32),
        pltpu.VMEM((_CH,), jnp.float32),
        pltpu.VMEM((16,), jnp.float32),
        pltpu.VMEM((16,), jnp.float32),
        pltpu.VMEM((256,), jnp.float32),
        pltpu.VMEM((256,), jnp.float32),
        pltpu.VMEM((32,), jnp.float32),
        pltpu.VMEM((32,), jnp.float32),
        pltpu.VMEM((16,), jnp.float32),
        pltpu.VMEM((16,), jnp.float32),
        pltpu.VMEM_SHARED((256,), jnp.float32),
        pltpu.VMEM_SHARED((256,), jnp.float32),
        pltpu.VMEM_SHARED((16,), jnp.float32),
        pltpu.SemaphoreType.DMA,
        pltpu.SemaphoreType.DMA,
        pltpu.SemaphoreType.DMA,
        pltpu.SemaphoreType.DMA,
        pltpu.SemaphoreType.REGULAR,
    ],
)
def _net_sc(table_hbm, idx_hbm, out_hbm, part_hbm,
            idx_v, val_v, buf, stage_m, stage_s, lm_v, ls_v,
            stage32, other32, stage_lse, lse_v,
            shared_m, shared_s, shared_lse,
            isem, gsem, csem0, csem1, xsem):
    c = lax.axis_index("c")
    s = lax.axis_index("s")
    wid = c * _NS + s

    icopy = pltpu.async_copy(idx_hbm.at[pl.ds(wid * _BPW, _BPW)], idx_v, isem)

    dma0 = pltpu.async_copy(
        table_hbm.at[pl.ds(wid * _CH, _SUBV * 16)],
        buf.at[pl.ds(0, _SUBV * 16)],
        csem0,
    )
    sub1_off = _SUBV * 16

    @pl.when(wid < 31)
    def _():
        pltpu.async_copy(
            table_hbm.at[pl.ds(wid * _CH + sub1_off, _CH - sub1_off)],
            buf.at[pl.ds(sub1_off, _CH - sub1_off)],
            csem1,
        )

    @pl.when(wid == 31)
    def _():
        pltpu.async_copy(
            table_hbm.at[pl.ds(_LAST_BASE + sub1_off, _LAST_SUB1)],
            buf.at[pl.ds(sub1_off, _LAST_SUB1)],
            csem1,
        )
        ninf = jnp.full((16,), -jnp.inf, jnp.float32)
        for k in range((_CH - _LAST_VALID) // 16):
            buf[pl.ds(_LAST_VALID + k * 16, 16)] = ninf

    icopy.wait()
    gathers = []
    for j in range(_IROWS):
        gathers.append(
            pltpu.async_copy(
                table_hbm.at[idx_v.at[pl.ds(j * 128, 128)]],
                val_v.at[pl.ds(j * 128, 128)],
                gsem,
            )
        )

    _p1 = jax.named_scope("phase_pass1"); _p1.__enter__()

    def max_body(i, accs, off=0):
        base = off + i * (_UNROLL * 16)
        return tuple(
            jnp.maximum(accs[j], buf[pl.ds(base + j * 16, 16)])
            for j in range(_UNROLL)
        )

    def sum_body(i, accs, off=0, m=None):
        base = off + i * (_UNROLL * 16)
        return tuple(
            accs[j] + jnp.exp(buf[pl.ds(base + j * 16, 16)] - m)
            for j in range(_UNROLL)
        )

    def _tree_max(accs):
        mv = accs[0]
        for j in range(1, _UNROLL):
            mv = jnp.maximum(mv, accs[j])
        return _bfly_max(mv)

    def _tree_sum(accs):
        sv = accs[0]
        for j in range(1, _UNROLL):
            sv = sv + accs[j]
        return _bfly_sum(sv)

    ninf = jnp.full((16,), -jnp.inf, jnp.float32)
    zero = jnp.zeros((16,), jnp.float32)

    dma0.wait()
    maccs = lax.fori_loop(0, _NBS, functools.partial(max_body, off=0),
                          (ninf,) * _UNROLL)
    m_a = jnp.maximum(_tree_max(maccs), buf[pl.ds(976 * 16, 16)])
    m_a = _bfly_max(m_a)
    saccs = lax.fori_loop(0, _NBS,
                          functools.partial(sum_body, off=0, m=m_a),
                          (zero,) * _UNROLL)
    s_a = _tree_sum(saccs) + _bfly_sum(jnp.exp(buf[pl.ds(976 * 16, 16)] - m_a))
    _p1.__exit__(None, None, None)

    _p2 = jax.named_scope("phase_pass2"); _p2.__enter__()

    @pl.when(wid < 31)
    def _():
        pltpu.make_async_copy(
            table_hbm.at[pl.ds(0, _CH - sub1_off)],
            buf.at[pl.ds(sub1_off, _CH - sub1_off)],
            csem1,
        ).wait()

    @pl.when(wid == 31)
    def _():
        pltpu.make_async_copy(
            table_hbm.at[pl.ds(0, _LAST_SUB1)],
            buf.at[pl.ds(sub1_off, _LAST_SUB1)],
            csem1,
        ).wait()

    maccs = lax.fori_loop(0, _NBS,
                          functools.partial(max_body, off=977 * 16),
                          (ninf,) * _UNROLL)
    m_b = jnp.maximum(_tree_max(maccs), buf[pl.ds(1953 * 16, 16)])
    m_b = _bfly_max(m_b)
    saccs = lax.fori_loop(0, _NBS,
                          functools.partial(sum_body, off=977 * 16, m=m_b),
                          (zero,) * _UNROLL)
    s_b = _tree_sum(saccs) + _bfly_sum(
        jnp.exp(buf[pl.ds(1953 * 16, 16)] - m_b))

    m_loc = jnp.maximum(m_a, m_b)
    w_a = jnp.where(m_a > -jnp.inf, jnp.exp(m_a - m_loc), 0.0)
    w_b = jnp.where(m_b > -jnp.inf, jnp.exp(m_b - m_loc), 0.0)
    s_loc = s_a * w_a + s_b * w_b
    _p2.__exit__(None, None, None)

    _p3 = jax.named_scope("phase_combine"); _p3.__enter__()
    stage_m[...] = m_loc
    stage_s[...] = s_loc
    pltpu.sync_copy(stage_m, shared_m.at[pl.ds(s * 16, 16)])
    pltpu.sync_copy(stage_s, shared_s.at[pl.ds(s * 16, 16)])
    plsc.subcore_barrier()

    @pl.when(s == 0)
    def _():
        pltpu.sync_copy(shared_m, lm_v)
        pltpu.sync_copy(shared_s, ls_v)
        m_c = lm_v[pl.ds(0, 16)]
        for r in range(1, 16):
            m_c = jnp.maximum(m_c, lm_v[pl.ds(r * 16, 16)])
        s_c = jnp.zeros((16,), jnp.float32)
        for r in range(16):
            s_c = s_c + ls_v[pl.ds(r * 16, 16)] * jnp.exp(
                lm_v[pl.ds(r * 16, 16)] - m_c)
        stage32[pl.ds(0, 16)] = m_c
        stage32[pl.ds(16, 16)] = s_c
        pltpu.sync_copy(stage32, part_hbm.at[c])
        pltpu.semaphore_signal(xsem, 1, core_index=1 - c)
        pltpu.semaphore_wait(xsem, 1)
        pltpu.sync_copy(part_hbm.at[1 - c], other32)
        m_o = other32[pl.ds(0, 16)]
        s_o = other32[pl.ds(16, 16)]
        m_g = jnp.maximum(m_c, m_o)
        s_vec = s_c * jnp.exp(m_c - m_g) + s_o * jnp.exp(m_o - m_g)
        bits = lax.bitcast_convert_type(s_vec, jnp.int32).astype(jnp.float32)
        y = (bits * (1.0 / 8388608.0) - 127.0) * _LN2
        for _ in range(3):
            y = y + s_vec * jnp.exp(-y) - 1.0
        stage_lse[...] = m_g + y
        pltpu.sync_copy(stage_lse, shared_lse)

    plsc.subcore_barrier()
    pltpu.sync_copy(shared_lse, lse_v)
    lse = lse_v[...]
    _p3.__exit__(None, None, None)

    _p4 = jax.named_scope("phase_finish"); _p4.__enter__()
    for g in gathers:
        g.wait()
    for k in range(_BPW // 16):
        val_v[pl.ds(k * 16, 16)] = val_v[pl.ds(k * 16, 16)] - lse
    pltpu.sync_copy(val_v, out_hbm.at[pl.ds(wid * _BPW, _BPW)])
    _p4.__exit__(None, None, None)


def kernel(state_index, state_logits):
    out, _ = _net_sc(state_logits, state_index.astype(jnp.int32))
    return out


# speedup vs baseline: 1.0111x; 1.0111x over previous
import functools

import jax
import jax.numpy as jnp
from jax import lax
from jax.experimental import pallas as pl
from jax.experimental.pallas import tpu as pltpu
from jax.experimental.pallas import tpu_sc as plsc

NUM_E = 1_000_000
B = 16384

_NC, _NS = 2, 16
_NW = _NC * _NS
_BPW = B // _NW
_IROWS = _BPW // 128

_UNROLL = 16
_CH = 31264
_NV = _CH // 16
_SUBV = 977
_NBS = 61
_LAST_BASE = 31 * _CH
_LAST_VALID = NUM_E - _LAST_BASE
_LAST_SUB1 = _LAST_VALID - _SUBV * 16

_LN2 = 0.6931471805599453

_sc_mesh = plsc.VectorSubcoreMesh(core_axis_name="c", subcore_axis_name="s")


def _bfly_max(x):
    for sh in (8, 4, 2, 1):
        perm = lax.iota(jnp.int32, 16) ^ sh
        x = jnp.maximum(x, x.at[perm].get(mode="promise_in_bounds"))
    return x


def _bfly_sum(x):
    for sh in (8, 4, 2, 1):
        perm = lax.iota(jnp.int32, 16) ^ sh
        x = x + x.at[perm].get(mode="promise_in_bounds")
    return x


@functools.partial(
    pl.kernel,
    mesh=_sc_mesh,
    out_type=(
        jax.ShapeDtypeStruct((B,), jnp.float32),
        jax.ShapeDtypeStruct((2, 32), jnp.float32),
    ),
    scratch_types=[
        pltpu.VMEM((_BPW,), jnp.int32),
        pltpu.VMEM((_BPW,), jnp.float32),
        pltpu.VMEM((_CH,), jnp.float32),
        pltpu.VMEM((16,), jnp.float32),
        pltpu.VMEM((16,), jnp.float32),
        pltpu.VMEM((256,), jnp.float32),
        pltpu.VMEM((256,), jnp.float32),
        pltpu.VMEM((32,), jnp.float32),
        pltpu.VMEM((32,), jnp.float32),
        pltpu.VMEM((16,), jnp.float32),
        pltpu.VMEM((16,), jnp.float32),
        pltpu.VMEM_SHARED((256,), jnp.float32),
        pltpu.VMEM_SHARED((256,), jnp.float32),
        pltpu.VMEM_SHARED((16,), jnp.float32),
        pltpu.SemaphoreType.DMA,
        pltpu.SemaphoreType.DMA,
        pltpu.SemaphoreType.DMA,
        pltpu.SemaphoreType.DMA,
        pltpu.SemaphoreType.REGULAR,
    ],
)
def _net_sc(table_hbm, idx_hbm, out_hbm, part_hbm,
            idx_v, val_v, buf, stage_m, stage_s, lm_v, ls_v,
            stage32, other32, stage_lse, lse_v,
            shared_m, shared_s, shared_lse,
            isem, gsem, csem0, csem1, xsem):
    c = lax.axis_index("c")
    s = lax.axis_index("s")
    wid = c * _NS + s

    icopy = pltpu.async_copy(idx_hbm.at[pl.ds(wid * _BPW, _BPW)], idx_v, isem)

    dma0 = pltpu.async_copy(
        table_hbm.at[pl.ds(wid * _CH, _SUBV * 16)],
        buf.at[pl.ds(0, _SUBV * 16)],
        csem0,
    )
    sub1_off = _SUBV * 16

    @pl.when(wid < 31)
    def _():
        pltpu.async_copy(
            table_hbm.at[pl.ds(wid * _CH + sub1_off, _CH - sub1_off)],
            buf.at[pl.ds(sub1_off, _CH - sub1_off)],
            csem1,
        )

    @pl.when(wid == 31)
    def _():
        pltpu.async_copy(
            table_hbm.at[pl.ds(_LAST_BASE + sub1_off, _LAST_SUB1)],
            buf.at[pl.ds(sub1_off, _LAST_SUB1)],
            csem1,
        )
        ninf = jnp.full((16,), -jnp.inf, jnp.float32)
        for k in range((_CH - _LAST_VALID) // 16):
            buf[pl.ds(_LAST_VALID + k * 16, 16)] = ninf

    icopy.wait()
    gathers = []
    for j in range(_IROWS):
        gathers.append(
            pltpu.async_copy(
                table_hbm.at[idx_v.at[pl.ds(j * 128, 128)]],
                val_v.at[pl.ds(j * 128, 128)],
                gsem,
            )
        )

    ninf = jnp.full((16,), -jnp.inf, jnp.float32)
    maccs = (ninf,) * _UNROLL

    def max_body(i, accs, off=0):
        base = off + i * (_UNROLL * 16)
        return tuple(
            jnp.maximum(accs[j], buf[pl.ds(base + j * 16, 16)])
            for j in range(_UNROLL)
        )

    dma0.wait()
    maccs = lax.fori_loop(0, _NBS, functools.partial(max_body, off=0), maccs)

    @pl.when(wid < 31)
    def _():
        pltpu.make_async_copy(
            table_hbm.at[pl.ds(0, _CH - sub1_off)],
            buf.at[pl.ds(sub1_off, _CH - sub1_off)],
            csem1,
        ).wait()

    @pl.when(wid == 31)
    def _():
        pltpu.make_async_copy(
            table_hbm.at[pl.ds(0, _LAST_SUB1)],
            buf.at[pl.ds(sub1_off, _LAST_SUB1)],
            csem1,
        ).wait()

    maccs = lax.fori_loop(0, _NBS,
                          functools.partial(max_body, off=976 * 16), maccs)
    for v in (1952, 1953):
        maccs = (jnp.maximum(maccs[0], buf[pl.ds(v * 16, 16)]),) + maccs[1:]
    mv = maccs[0]
    for j in range(1, _UNROLL):
        mv = jnp.maximum(mv, maccs[j])
    m_loc = _bfly_max(mv)

    zero = jnp.zeros((16,), jnp.float32)
    saccs = (zero,) * _UNROLL

    def sum_body(i, accs):
        base = i * (_UNROLL * 16)
        return tuple(
            accs[j] + jnp.exp(buf[pl.ds(base + j * 16, 16)] - m_loc)
            for j in range(_UNROLL)
        )

    saccs = lax.fori_loop(0, 2 * _NBS, sum_body, saccs)
    for v in (1952, 1953):
        saccs = (saccs[0] + jnp.exp(buf[pl.ds(v * 16, 16)] - m_loc),
                 ) + saccs[1:]
    sv = saccs[0]
    for j in range(1, _UNROLL):
        sv = sv + saccs[j]
    s_loc = _bfly_sum(sv)

    stage_m[...] = m_loc
    stage_s[...] = s_loc
    pltpu.sync_copy(stage_m, shared_m.at[pl.ds(s * 16, 16)])
    pltpu.sync_copy(stage_s, shared_s.at[pl.ds(s * 16, 16)])
    plsc.subcore_barrier()

    @pl.when(s == 0)
    def _():
        pltpu.sync_copy(shared_m, lm_v)
        pltpu.sync_copy(shared_s, ls_v)
        m_c = lm_v[pl.ds(0, 16)]
        for r in range(1, 16):
            m_c = jnp.maximum(m_c, lm_v[pl.ds(r * 16, 16)])
        s_c = jnp.zeros((16,), jnp.float32)
        for r in range(16):
            s_c = s_c + ls_v[pl.ds(r * 16, 16)] * jnp.exp(
                lm_v[pl.ds(r * 16, 16)] - m_c)
        stage32[pl.ds(0, 16)] = m_c
        stage32[pl.ds(16, 16)] = s_c
        pltpu.sync_copy(stage32, part_hbm.at[c])
        pltpu.semaphore_signal(xsem, 1, core_index=1 - c)
        pltpu.semaphore_wait(xsem, 1)
        pltpu.sync_copy(part_hbm.at[1 - c], other32)
        m_o = other32[pl.ds(0, 16)]
        s_o = other32[pl.ds(16, 16)]
        m_g = jnp.maximum(m_c, m_o)
        s_vec = s_c * jnp.exp(m_c - m_g) + s_o * jnp.exp(m_o - m_g)
        bits = lax.bitcast_convert_type(s_vec, jnp.int32).astype(jnp.float32)
        y = (bits * (1.0 / 8388608.0) - 127.0) * _LN2
        for _ in range(3):
            y = y + s_vec * jnp.exp(-y) - 1.0
        stage_lse[...] = m_g + y
        pltpu.sync_copy(stage_lse, shared_lse)

    plsc.subcore_barrier()
    pltpu.sync_copy(shared_lse, lse_v)
    lse = lse_v[...]

    for g in gathers:
        g.wait()
    for k in range(_BPW // 16):
        val_v[pl.ds(k * 16, 16)] = val_v[pl.ds(k * 16, 16)] - lse
    pltpu.sync_copy(val_v, out_hbm.at[pl.ds(wid * _BPW, _BPW)])


def kernel(state_index, state_logits):
    out, _ = _net_sc(state_logits, state_index.astype(jnp.int32))
    return out
